# Initial kernel scaffold; baseline (speedup 1.0000x reference)
import jax
import jax.numpy as jnp
from jax.experimental import pallas as pl


def kernel(x1, x2, deconv_w, deconv_b, conv_w, conv_b, bn_gamma, bn_beta, bn_mean, bn_var):
    raise NotImplementedError("write your pallas kernel here")


# speedup vs baseline: 1.9609x; 1.9609x over previous
import functools

import jax
import jax.numpy as jnp
from jax.experimental import pallas as pl
from jax.experimental.pallas import tpu as pltpu


def _fused_up_kernel(xm_ref, xh_ref, sm_ref, sh_ref,
                     wdec_ref, bdec_ref, w1_ref, w2_ref, bias_ref, out_ref):
    TH = xm_ref.shape[1]
    W = xm_ref.shape[2] - 2
    cin = xm_ref.shape[3]
    Wo = 2 * W
    cmid = sm_ref.shape[3]
    t = pl.program_id(1)
    nT = pl.num_programs(1)

    xw = jnp.concatenate([xm_ref[0], xh_ref[0]], axis=0)

    M1 = (TH + 1) * W
    halves = []
    for pw in range(2):
        patch = jnp.concatenate(
            [xw[dh:dh + TH + 1, pw + dw:pw + dw + W, :].reshape(M1, cin)
             for dh in range(2) for dw in range(2)], axis=-1)
        acc = jnp.dot(patch, wdec_ref[pw],
                      preferred_element_type=jnp.float32)
        halves.append(jnp.maximum(acc + bdec_ref[...], 0.0))

    e0 = halves[0][:, :cmid].reshape(TH + 1, W, cmid)
    o0 = halves[0][:, cmid:].reshape(TH + 1, W, cmid)
    e1 = halves[1][:, :cmid].reshape(TH + 1, W, cmid)
    o1 = halves[1][:, cmid:].reshape(TH + 1, W, cmid)

    e_full = jnp.stack([e0, e1], axis=2).reshape(TH + 1, Wo, cmid)
    o_full = jnp.stack([o0, o1], axis=2).reshape(TH + 1, Wo, cmid)
    y = jnp.stack([o_full, e_full], axis=1).reshape(2 * TH + 2, Wo, cmid)

    ridx = jax.lax.broadcasted_iota(jnp.int32, (2 * TH + 2, Wo, cmid), 0)
    bad = ((t == 0) & (ridx == 0)) | ((t == nT - 1) & (ridx == 2 * TH + 1))
    y = jnp.where(bad, 0.0, y).astype(jnp.bfloat16)

    zcol = jnp.zeros((2 * TH + 2, 1, cmid), jnp.bfloat16)
    ypad = jnp.concatenate([zcol, y, zcol], axis=1)

    sw = jnp.concatenate([sm_ref[0], sh_ref[0]], axis=0)
    M2 = 2 * TH * Wo

    def im2col(a):
        cols = [a[r:r + 2 * TH, s:s + Wo, :].reshape(M2, cmid)
                for r in range(3) for s in range(3)]
        return jnp.concatenate(cols, axis=-1)

    acc = jnp.dot(im2col(ypad), w1_ref[...],
                  preferred_element_type=jnp.float32)
    acc = acc + jnp.dot(im2col(sw), w2_ref[...],
                        preferred_element_type=jnp.float32)
    out_ref[0] = jnp.maximum(acc + bias_ref[...], 0.0).astype(out_ref.dtype)


def _row_tile(h, max_tile=8):
    d = max_tile - max_tile % 2
    while d >= 2:
        if h % d == 0:
            return d
        d -= 2
    return h


@functools.partial(jax.jit, static_argnames=())
def kernel(x1, x2, deconv_w, deconv_b, conv_w, conv_b,
           bn_gamma, bn_beta, bn_mean, bn_var):
    bn_eps = 1e-5
    N, Cin, H, W = x1.shape
    Cmid = deconv_w.shape[1]
    Cout = conv_w.shape[0]
    Ho, Wo = 2 * H, 2 * W
    dt = x1.dtype

    xpad = jnp.pad(jnp.transpose(x1, (0, 2, 3, 1)),
                   ((0, 0), (1, 1), (1, 1), (0, 0))).astype(jnp.bfloat16)
    spad = jnp.pad(jnp.transpose(x2, (0, 2, 3, 1)),
                   ((0, 0), (1, 1), (1, 1), (0, 0))).astype(jnp.bfloat16)

    wf = jnp.transpose(jnp.flip(deconv_w, axis=(2, 3)), (2, 3, 0, 1))
    wdec = jnp.stack([
        jnp.concatenate([
            jnp.concatenate([wf[2 * dh, pw + 2 * dw] for dh in range(2)
                             for dw in range(2)], axis=0),
            jnp.concatenate([wf[2 * dh + 1, pw + 2 * dw] for dh in range(2)
                             for dw in range(2)], axis=0)], axis=1)
        for pw in range(2)], axis=0).astype(jnp.bfloat16)
    bdec = jnp.concatenate([deconv_b, deconv_b]).reshape(1, 2 * Cmid)
    bdec = bdec.astype(jnp.float32)

    scale = bn_gamma * jax.lax.rsqrt(bn_var + bn_eps)
    w_eff = conv_w * scale[:, None, None, None]
    bias_eff = (bn_beta + scale * (conv_b - bn_mean)).reshape(1, Cout)
    bias_eff = bias_eff.astype(jnp.float32)
    w_t = jnp.transpose(w_eff, (2, 3, 1, 0))
    w1 = w_t[:, :, :Cmid, :].reshape(9 * Cmid, Cout).astype(jnp.bfloat16)
    w2 = w_t[:, :, Cmid:, :].reshape(9 * Cmid, Cout).astype(jnp.bfloat16)

    TH = _row_tile(H)
    nT = H // TH
    Wp, Wop = W + 2, Wo + 2

    out_flat = pl.pallas_call(
        _fused_up_kernel,
        out_shape=jax.ShapeDtypeStruct((N, Ho * Wo, Cout), dt),
        grid_spec=pltpu.PrefetchScalarGridSpec(
            num_scalar_prefetch=0,
            grid=(N, nT),
            in_specs=[
                pl.BlockSpec((1, TH, Wp, Cin), lambda n, t: (n, t, 0, 0)),
                pl.BlockSpec((1, 2, Wp, Cin),
                             lambda n, t: (n, t * (TH // 2) + TH // 2, 0, 0)),
                pl.BlockSpec((1, 2 * TH, Wop, Cmid), lambda n, t: (n, t, 0, 0)),
                pl.BlockSpec((1, 2, Wop, Cmid), lambda n, t: (n, (t + 1) * TH, 0, 0)),
                pl.BlockSpec((2, 4 * Cin, 2 * Cmid), lambda n, t: (0, 0, 0)),
                pl.BlockSpec((1, 2 * Cmid), lambda n, t: (0, 0)),
                pl.BlockSpec((9 * Cmid, Cout), lambda n, t: (0, 0)),
                pl.BlockSpec((9 * Cmid, Cout), lambda n, t: (0, 0)),
                pl.BlockSpec((1, Cout), lambda n, t: (0, 0)),
            ],
            out_specs=pl.BlockSpec((1, 2 * TH * Wo, Cout), lambda n, t: (n, t, 0)),
        ),
        compiler_params=pltpu.CompilerParams(
            dimension_semantics=("parallel", "parallel"),
            vmem_limit_bytes=100 * 1024 * 1024,
        ),
    )(xpad, xpad, spad, spad, wdec, bdec, w1, w2, bias_eff)

    out = out_flat.reshape(N, Ho, Wo, Cout)
    return jnp.transpose(out, (0, 3, 1, 2))
